# baseline (speedup 1.0000x reference)
import functools

import jax
import jax.numpy as jnp
from jax import lax
from jax.experimental import pallas as pl
from jax.experimental.pallas import tpu as pltpu
from jax.experimental.pallas import tpu_sc as plsc

B, C, H, W = 2, 96, 512, 512
N, K = 8192, 9
HW = H * W
NTAP = 4 * K
CV = C // 16
CP = 128
MOX = 16.0 / W
MOY = 16.0 / H

NC, NS = 2, 16
NW = NC * NS
NPB = N // NW

_HIGH = lax.Precision.HIGHEST


_HB = 8


def _transpose_body(x_ref, o_ref):
    for r in range(_HB):
        o_ref[r, :, 0:C] = x_ref[0, :, r, :].T


def _transpose(feat, b):
    return pl.pallas_call(
        _transpose_body,
        grid=(H // _HB,),
        in_specs=[pl.BlockSpec((1, C, _HB, W), lambda h, _b=b: (_b, 0, h, 0))],
        out_specs=pl.BlockSpec((_HB, W, CP), lambda h: (h, 0, 0)),
        out_shape=jax.ShapeDtypeStruct((H, W, CP), jnp.float32),
    )(feat)


_ACH = 16
_ANCH = NPB // _ACH


def _sc1_body(feat_hbm, xs_hbm, ys_hbm, fa_hbm, xv, yv, idxb, rows,
              outv, sem):
    wid = lax.axis_index("s") * NC + lax.axis_index("c")
    base = wid * NPB
    pltpu.sync_copy(xs_hbm.at[pl.ds(base, NPB)], xv)
    pltpu.sync_copy(ys_hbm.at[pl.ds(base, NPB)], yv)

    def chunk(i, carry):
        px = xv[pl.ds(i * _ACH, 16)]
        py = yv[pl.ds(i * _ACH, 16)]
        fx = (px + 1.0) * 0.5 * (W - 1)
        fy = (py + 1.0) * 0.5 * (H - 1)
        x0 = fx.astype(jnp.int32)
        y0 = fy.astype(jnp.int32)
        wx1 = fx - x0.astype(jnp.float32)
        wy1 = fy - y0.astype(jnp.float32)
        wx0 = 1.0 - wx1
        wy0 = 1.0 - wy1
        x1 = jnp.minimum(x0 + 1, W - 1)
        y1 = jnp.minimum(y0 + 1, H - 1)
        r0 = y0 * W
        r1 = y1 * W
        idxb[pl.ds(0, 16)] = r0 + x0
        idxb[pl.ds(16, 16)] = r0 + x1
        idxb[pl.ds(32, 16)] = r1 + x0
        idxb[pl.ds(48, 16)] = r1 + x1
        w00 = wy0 * wx0
        w01 = wy0 * wx1
        w10 = wy1 * wx0
        w11 = wy1 * wx1
        pltpu.async_copy(feat_hbm.at[idxb], rows, sem).wait()

        for p in range(_ACH):
            w0 = w00[p]
            w1 = w01[p]
            w2 = w10[p]
            w3 = w11[p]
            for cv in range(CV):
                sl = pl.ds(cv * 16, 16)
                outv[i * _ACH + p, sl] = (
                    w0 * rows[p, sl] + w1 * rows[16 + p, sl]
                    + w2 * rows[32 + p, sl] + w3 * rows[48 + p, sl])
        return carry

    lax.fori_loop(0, _ANCH, chunk, 0)
    pltpu.sync_copy(outv, fa_hbm.at[pl.ds(base, NPB)])


@functools.cache
def _sc_anchor_kernel():
    return pl.kernel(
        _sc1_body,
        out_type=jax.ShapeDtypeStruct((N, C), jnp.float32),
        mesh=plsc.VectorSubcoreMesh(core_axis_name="c", subcore_axis_name="s",
                                    num_cores=NC, num_subcores=NS),
        scratch_types=[
            pltpu.VMEM((NPB,), jnp.float32),
            pltpu.VMEM((NPB,), jnp.float32),
            pltpu.VMEM((4 * _ACH,), jnp.int32),
            pltpu.VMEM((4 * _ACH, CP), jnp.float32),
            pltpu.VMEM((NPB, C), jnp.float32),
            pltpu.SemaphoreType.DMA,
        ],
    )


_RB = 2048


def _mlp_body(x_ref, xs_ref, ys_ref, w1_ref, b1_ref, wr_ref, br_ref, w2_ref,
              b2_ref, idx_ref, wts_ref):
    x = x_ref[...]
    h = jnp.dot(x, w1_ref[...], precision=_HIGH) + b1_ref[...]
    h = jnp.where(h >= 0, h, 0.2 * h)
    h2 = h + jnp.dot(h, wr_ref[...], precision=_HIGH) + br_ref[...]
    h2 = jnp.where(h2 >= 0, h2, 0.2 * h2)
    r = jnp.dot(h2, w2_ref[...], precision=_HIGH) + b2_ref[...]

    rt = r.T
    xo = jnp.tanh(rt[0:9]) * MOX
    yo = jnp.tanh(rt[9:18]) * MOY
    wl = rt[18:27]
    m = jnp.max(wl, axis=0, keepdims=True)
    e = jnp.exp(wl - m)
    dynw = e / jnp.sum(e, axis=0, keepdims=True)

    cx = xs_ref[...]
    cy = ys_ref[...]
    fx = (cx + xo + 1.0) * 0.5 * (W - 1)
    fy = (cy + yo + 1.0) * 0.5 * (H - 1)
    x0f = jnp.floor(fx)
    y0f = jnp.floor(fy)
    wx1 = fx - x0f
    wy1 = fy - y0f
    wx0 = 1.0 - wx1
    wy0 = 1.0 - wy1
    x0 = jnp.clip(x0f.astype(jnp.int32), 0, W - 1)
    x1 = jnp.clip(x0f.astype(jnp.int32) + 1, 0, W - 1)
    y0 = jnp.clip(y0f.astype(jnp.int32), 0, H - 1)
    y1 = jnp.clip(y0f.astype(jnp.int32) + 1, 0, H - 1)

    r0 = y0 * W
    r1 = y1 * W
    idx_ref[...] = jnp.concatenate(
        [r0 + x0, r0 + x1, r1 + x0, r1 + x1], axis=0).T
    wts_ref[...] = jnp.concatenate(
        [dynw * wy0 * wx0, dynw * wy0 * wx1,
         dynw * wy1 * wx0, dynw * wy1 * wx1], axis=0).T


def _mlp(xin, xs2, ys2, w1t, b1, wrt, br, w2t, b2p):
    return pl.pallas_call(
        _mlp_body,
        grid=(N // _RB,),
        in_specs=[
            pl.BlockSpec((_RB, 128), lambda g: (g, 0)),
            pl.BlockSpec((1, _RB), lambda g: (0, g)),
            pl.BlockSpec((1, _RB), lambda g: (0, g)),
            pl.BlockSpec((128, 64), lambda g: (0, 0)),
            pl.BlockSpec((1, 64), lambda g: (0, 0)),
            pl.BlockSpec((64, 64), lambda g: (0, 0)),
            pl.BlockSpec((1, 64), lambda g: (0, 0)),
            pl.BlockSpec((64, 32), lambda g: (0, 0)),
            pl.BlockSpec((1, 32), lambda g: (0, 0)),
        ],
        out_specs=[
            pl.BlockSpec((_RB, NTAP), lambda g: (g, 0)),
            pl.BlockSpec((_RB, NTAP), lambda g: (g, 0)),
        ],
        out_shape=[
            jax.ShapeDtypeStruct((N, NTAP), jnp.int32),
            jax.ShapeDtypeStruct((N, NTAP), jnp.float32),
        ],
    )(xin, xs2, ys2, w1t, b1, wrt, br, w2t, b2p)


_PC = 2
_RING = 8
_SEC = 2
_PTS_S = NPB // _SEC
_NCH_S = _PTS_S // _PC


def _sc2_body(feat_hbm, idx_hbm, wts_hbm, out_hbm, idxv, wtsv, rows, outv,
              *sems):
    wid = lax.axis_index("s") * NC + lax.axis_index("c")
    base = wid * NPB

    def start(ch, slot):
        pltpu.async_copy(
            feat_hbm.at[idxv.at[pl.ds(ch * (_PC * NTAP), _PC * NTAP)]],
            rows.at[slot], sems[slot])

    def wait(ch, slot):
        pltpu.make_async_copy(
            feat_hbm.at[idxv.at[pl.ds(ch * (_PC * NTAP), _PC * NTAP)]],
            rows.at[slot], sems[slot]).wait()

    def section(sct, carry):
        sbase = (base + sct * _PTS_S) * NTAP
        pltpu.sync_copy(idx_hbm.at[pl.ds(sbase, _PTS_S * NTAP)], idxv)
        pltpu.sync_copy(wts_hbm.at[pl.ds(sbase, _PTS_S * NTAP)], wtsv)
        for r in range(_RING):
            start(r, r)

        def group(g, carry2):
            for r in range(_RING):
                ch = g * _RING + r
                wait(ch, r)
                for p in range(_PC):
                    ptl = ch * _PC + p
                    o = ptl * NTAP
                    wa = wtsv[pl.ds(o, 16)]
                    wb = wtsv[pl.ds(o + 16, 16)]
                    wc = wtsv[pl.ds(o + 20, 16)]
                    acc = [None] * CV
                    for j in range(NTAP):
                        if j < 16:
                            wj = wa[j]
                        elif j < 32:
                            wj = wb[j - 16]
                        else:
                            wj = wc[j - 20]
                        for cv in range(CV):
                            t = wj * rows[r, p * NTAP + j, pl.ds(cv * 16, 16)]
                            acc[cv] = t if acc[cv] is None else acc[cv] + t
                    for cv in range(CV):
                        outv[sct * _PTS_S + ptl, pl.ds(cv * 16, 16)] = acc[cv]

                @pl.when(ch + _RING < _NCH_S)
                def _():
                    start(ch + _RING, r)
            return carry2

        lax.fori_loop(0, _NCH_S // _RING, group, 0)
        return carry

    lax.fori_loop(0, _SEC, section, 0)
    pltpu.sync_copy(outv, out_hbm.at[pl.ds(base, NPB)])


@functools.cache
def _sc_deform_kernel():
    return pl.kernel(
        _sc2_body,
        out_type=jax.ShapeDtypeStruct((N, C), jnp.float32),
        mesh=plsc.VectorSubcoreMesh(core_axis_name="c", subcore_axis_name="s",
                                    num_cores=NC, num_subcores=NS),
        scratch_types=[
            pltpu.VMEM((_PTS_S * NTAP,), jnp.int32),
            pltpu.VMEM((_PTS_S * NTAP,), jnp.float32),
            pltpu.VMEM((_RING, _PC * NTAP, CP), jnp.float32),
            pltpu.VMEM((NPB, C), jnp.float32),
        ] + [pltpu.SemaphoreType.DMA] * _RING,
    )


_PERM = tuple(range(0, 2 * K, 2)) + tuple(range(1, 2 * K, 2)) \
    + tuple(range(2 * K, 3 * K))


def kernel(feat_map, coords_2d, W1, b1, Wr, br, W2, b2):
    w1t = jnp.pad(W1, ((0, 0), (0, 30))).T
    perm = jnp.array(_PERM, dtype=jnp.int32)
    w2t = jnp.pad(W2[perm], ((0, 5), (0, 0))).T
    b2p = jnp.pad(b2[perm], (0, 5)).reshape(1, 32)
    b1r = b1.reshape(1, 64)
    brr = br.reshape(1, 64)
    wrt = Wr.T

    featT = [None] * B
    outs = [None] * B
    fas = [None] * B
    for b in range(B):
        featT[b] = _transpose(feat_map, b).reshape(HW, CP)
    sc_prev = None
    for b in range(B):
        xy = coords_2d[b]
        xs = xy[:, 0]
        ys = xy[:, 1]
        ft = featT[b]
        if sc_prev is not None:
            ft, _ = lax.optimization_barrier((ft, sc_prev))
        fas[b] = _sc_anchor_kernel()(ft, xs, ys)
        sc_prev = fas[b]
    for b in range(B):
        xy = coords_2d[b]
        xs = xy[:, 0]
        ys = xy[:, 1]
        xin = jnp.concatenate(
            [fas[b], xy, jnp.zeros((N, 30), jnp.float32)], axis=1)
        idx, wts = _mlp(xin, xs.reshape(1, N), ys.reshape(1, N),
                        w1t, b1r, wrt, brr, w2t, b2p)
        idxf = idx.reshape(-1)
        if sc_prev is not None:
            idxf, _ = lax.optimization_barrier((idxf, sc_prev))
        outs[b] = _sc_deform_kernel()(featT[b], idxf, wts.reshape(-1))
        sc_prev = outs[b]
    return jnp.stack(outs)


# speedup vs baseline: 1.0789x; 1.0789x over previous
import functools

import jax
import jax.numpy as jnp
from jax import lax
from jax.experimental import pallas as pl
from jax.experimental.pallas import tpu as pltpu
from jax.experimental.pallas import tpu_sc as plsc

B, C, H, W = 2, 96, 512, 512
N, K = 8192, 9
HW = H * W
NTAP = 4 * K
CV = C // 16
CP = 128
MOX = 16.0 / W
MOY = 16.0 / H

NC, NS = 2, 16
NW = NC * NS
NPB = N // NW

_HIGH = lax.Precision.HIGHEST


_HB = 16


def _transpose_body(x_ref, o_ref):
    for r in range(_HB):
        o_ref[r, :, 0:C] = x_ref[0, :, r, :].T


def _transpose(feat, b):
    return pl.pallas_call(
        _transpose_body,
        grid=(H // _HB,),
        in_specs=[pl.BlockSpec((1, C, _HB, W), lambda h, _b=b: (_b, 0, h, 0))],
        out_specs=pl.BlockSpec((_HB, W, CP), lambda h: (h, 0, 0)),
        out_shape=jax.ShapeDtypeStruct((H, W, CP), jnp.float32),
    )(feat)


_ACH = 16
_ANCH = NPB // _ACH


def _sc1_body(feat_hbm, xs_hbm, ys_hbm, fa_hbm, xv, yv, idxb, rows,
              outv, sem):
    wid = lax.axis_index("s") * NC + lax.axis_index("c")
    base = wid * NPB
    pltpu.sync_copy(xs_hbm.at[pl.ds(base, NPB)], xv)
    pltpu.sync_copy(ys_hbm.at[pl.ds(base, NPB)], yv)

    def chunk(i, carry):
        px = xv[pl.ds(i * _ACH, 16)]
        py = yv[pl.ds(i * _ACH, 16)]
        fx = (px + 1.0) * 0.5 * (W - 1)
        fy = (py + 1.0) * 0.5 * (H - 1)
        x0 = fx.astype(jnp.int32)
        y0 = fy.astype(jnp.int32)
        wx1 = fx - x0.astype(jnp.float32)
        wy1 = fy - y0.astype(jnp.float32)
        wx0 = 1.0 - wx1
        wy0 = 1.0 - wy1
        x1 = jnp.minimum(x0 + 1, W - 1)
        y1 = jnp.minimum(y0 + 1, H - 1)
        r0 = y0 * W
        r1 = y1 * W
        idxb[pl.ds(0, 16)] = r0 + x0
        idxb[pl.ds(16, 16)] = r0 + x1
        idxb[pl.ds(32, 16)] = r1 + x0
        idxb[pl.ds(48, 16)] = r1 + x1
        w00 = wy0 * wx0
        w01 = wy0 * wx1
        w10 = wy1 * wx0
        w11 = wy1 * wx1
        pltpu.async_copy(feat_hbm.at[idxb], rows, sem).wait()

        for p in range(_ACH):
            w0 = w00[p]
            w1 = w01[p]
            w2 = w10[p]
            w3 = w11[p]
            for cv in range(CV):
                sl = pl.ds(cv * 16, 16)
                outv[i * _ACH + p, sl] = (
                    w0 * rows[p, sl] + w1 * rows[16 + p, sl]
                    + w2 * rows[32 + p, sl] + w3 * rows[48 + p, sl])
        return carry

    lax.fori_loop(0, _ANCH, chunk, 0)
    pltpu.sync_copy(outv, fa_hbm.at[pl.ds(base, NPB)])


@functools.cache
def _sc_anchor_kernel():
    return pl.kernel(
        _sc1_body,
        out_type=jax.ShapeDtypeStruct((N, C), jnp.float32),
        mesh=plsc.VectorSubcoreMesh(core_axis_name="c", subcore_axis_name="s",
                                    num_cores=NC, num_subcores=NS),
        scratch_types=[
            pltpu.VMEM((NPB,), jnp.float32),
            pltpu.VMEM((NPB,), jnp.float32),
            pltpu.VMEM((4 * _ACH,), jnp.int32),
            pltpu.VMEM((4 * _ACH, CP), jnp.float32),
            pltpu.VMEM((NPB, C), jnp.float32),
            pltpu.SemaphoreType.DMA,
        ],
    )


_RB = 2048


def _mlp_body(x_ref, xs_ref, ys_ref, w1_ref, b1_ref, wr_ref, br_ref, w2_ref,
              b2_ref, idx_ref, wts_ref):
    x = x_ref[...]
    h = jnp.dot(x, w1_ref[...], precision=_HIGH) + b1_ref[...]
    h = jnp.where(h >= 0, h, 0.2 * h)
    h2 = h + jnp.dot(h, wr_ref[...], precision=_HIGH) + br_ref[...]
    h2 = jnp.where(h2 >= 0, h2, 0.2 * h2)
    r = jnp.dot(h2, w2_ref[...], precision=_HIGH) + b2_ref[...]

    rt = r.T
    xo = jnp.tanh(rt[0:9]) * MOX
    yo = jnp.tanh(rt[9:18]) * MOY
    wl = rt[18:27]
    m = jnp.max(wl, axis=0, keepdims=True)
    e = jnp.exp(wl - m)
    dynw = e / jnp.sum(e, axis=0, keepdims=True)

    cx = xs_ref[...]
    cy = ys_ref[...]
    fx = (cx + xo + 1.0) * 0.5 * (W - 1)
    fy = (cy + yo + 1.0) * 0.5 * (H - 1)
    x0f = jnp.floor(fx)
    y0f = jnp.floor(fy)
    wx1 = fx - x0f
    wy1 = fy - y0f
    wx0 = 1.0 - wx1
    wy0 = 1.0 - wy1
    x0 = jnp.clip(x0f.astype(jnp.int32), 0, W - 1)
    x1 = jnp.clip(x0f.astype(jnp.int32) + 1, 0, W - 1)
    y0 = jnp.clip(y0f.astype(jnp.int32), 0, H - 1)
    y1 = jnp.clip(y0f.astype(jnp.int32) + 1, 0, H - 1)

    r0 = y0 * W
    r1 = y1 * W
    idx_ref[...] = jnp.concatenate(
        [r0 + x0, r0 + x1, r1 + x0, r1 + x1], axis=0).T
    wts_ref[...] = jnp.concatenate(
        [dynw * wy0 * wx0, dynw * wy0 * wx1,
         dynw * wy1 * wx0, dynw * wy1 * wx1], axis=0).T


def _mlp(xin, xs2, ys2, w1t, b1, wrt, br, w2t, b2p):
    return pl.pallas_call(
        _mlp_body,
        grid=(N // _RB,),
        in_specs=[
            pl.BlockSpec((_RB, 128), lambda g: (g, 0)),
            pl.BlockSpec((1, _RB), lambda g: (0, g)),
            pl.BlockSpec((1, _RB), lambda g: (0, g)),
            pl.BlockSpec((128, 64), lambda g: (0, 0)),
            pl.BlockSpec((1, 64), lambda g: (0, 0)),
            pl.BlockSpec((64, 64), lambda g: (0, 0)),
            pl.BlockSpec((1, 64), lambda g: (0, 0)),
            pl.BlockSpec((64, 32), lambda g: (0, 0)),
            pl.BlockSpec((1, 32), lambda g: (0, 0)),
        ],
        out_specs=[
            pl.BlockSpec((_RB, NTAP), lambda g: (g, 0)),
            pl.BlockSpec((_RB, NTAP), lambda g: (g, 0)),
        ],
        out_shape=[
            jax.ShapeDtypeStruct((N, NTAP), jnp.int32),
            jax.ShapeDtypeStruct((N, NTAP), jnp.float32),
        ],
    )(xin, xs2, ys2, w1t, b1, wrt, br, w2t, b2p)


_PC = 2
_RING = 4
_SEC = 2
_PTS_S = NPB // _SEC
_NCH_S = _PTS_S // _PC


def _sc2_body(feat_hbm, idx_hbm, wts_hbm, out_hbm, idxv, wtsv, rows, outv,
              *sems):
    wid = lax.axis_index("s") * NC + lax.axis_index("c")
    base = wid * NPB

    def start(ch, slot):
        pltpu.async_copy(
            feat_hbm.at[idxv.at[pl.ds(ch * (_PC * NTAP), _PC * NTAP)]],
            rows.at[slot], sems[slot])

    def wait(ch, slot):
        pltpu.make_async_copy(
            feat_hbm.at[idxv.at[pl.ds(ch * (_PC * NTAP), _PC * NTAP)]],
            rows.at[slot], sems[slot]).wait()

    def section(sct, carry):
        sbase = (base + sct * _PTS_S) * NTAP
        pltpu.sync_copy(idx_hbm.at[pl.ds(sbase, _PTS_S * NTAP)], idxv)
        pltpu.sync_copy(wts_hbm.at[pl.ds(sbase, _PTS_S * NTAP)], wtsv)
        for r in range(_RING):
            start(r, r)

        def group(g, carry2):
            for r in range(_RING):
                ch = g * _RING + r
                wait(ch, r)
                for p in range(_PC):
                    ptl = ch * _PC + p
                    o = ptl * NTAP
                    wa = wtsv[pl.ds(o, 16)]
                    wb = wtsv[pl.ds(o + 16, 16)]
                    wc = wtsv[pl.ds(o + 20, 16)]
                    acc = [None] * CV
                    for j in range(NTAP):
                        if j < 16:
                            wj = wa[j]
                        elif j < 32:
                            wj = wb[j - 16]
                        else:
                            wj = wc[j - 20]
                        for cv in range(CV):
                            t = wj * rows[r, p * NTAP + j, pl.ds(cv * 16, 16)]
                            acc[cv] = t if acc[cv] is None else acc[cv] + t
                    for cv in range(CV):
                        outv[sct * _PTS_S + ptl, pl.ds(cv * 16, 16)] = acc[cv]

                @pl.when(ch + _RING < _NCH_S)
                def _():
                    start(ch + _RING, r)
            return carry2

        lax.fori_loop(0, _NCH_S // _RING, group, 0)
        return carry

    lax.fori_loop(0, _SEC, section, 0)
    pltpu.sync_copy(outv, out_hbm.at[pl.ds(base, NPB)])


@functools.cache
def _sc_deform_kernel():
    return pl.kernel(
        _sc2_body,
        out_type=jax.ShapeDtypeStruct((N, C), jnp.float32),
        mesh=plsc.VectorSubcoreMesh(core_axis_name="c", subcore_axis_name="s",
                                    num_cores=NC, num_subcores=NS),
        scratch_types=[
            pltpu.VMEM((_PTS_S * NTAP,), jnp.int32),
            pltpu.VMEM((_PTS_S * NTAP,), jnp.float32),
            pltpu.VMEM((_RING, _PC * NTAP, CP), jnp.float32),
            pltpu.VMEM((NPB, C), jnp.float32),
        ] + [pltpu.SemaphoreType.DMA] * _RING,
    )


_PERM = tuple(range(0, 2 * K, 2)) + tuple(range(1, 2 * K, 2)) \
    + tuple(range(2 * K, 3 * K))


def kernel(feat_map, coords_2d, W1, b1, Wr, br, W2, b2):
    w1t = jnp.pad(W1, ((0, 0), (0, 30))).T
    perm = jnp.array(_PERM, dtype=jnp.int32)
    w2t = jnp.pad(W2[perm], ((0, 5), (0, 0))).T
    b2p = jnp.pad(b2[perm], (0, 5)).reshape(1, 32)
    b1r = b1.reshape(1, 64)
    brr = br.reshape(1, 64)
    wrt = Wr.T

    featT = [None] * B
    outs = [None] * B
    fas = [None] * B
    for b in range(B):
        featT[b] = _transpose(feat_map, b).reshape(HW, CP)
    sc_prev = None
    for b in range(B):
        xy = coords_2d[b]
        xs = xy[:, 0]
        ys = xy[:, 1]
        ft = featT[b]
        if sc_prev is not None:
            ft, _ = lax.optimization_barrier((ft, sc_prev))
        fas[b] = _sc_anchor_kernel()(ft, xs, ys)
        sc_prev = fas[b]
    for b in range(B):
        xy = coords_2d[b]
        xs = xy[:, 0]
        ys = xy[:, 1]
        xin = jnp.concatenate(
            [fas[b], xy, jnp.zeros((N, 30), jnp.float32)], axis=1)
        idx, wts = _mlp(xin, xs.reshape(1, N), ys.reshape(1, N),
                        w1t, b1r, wrt, brr, w2t, b2p)
        idxf = idx.reshape(-1)
        if sc_prev is not None:
            idxf, _ = lax.optimization_barrier((idxf, sc_prev))
        outs[b] = _sc_deform_kernel()(featT[b], idxf, wts.reshape(-1))
        sc_prev = outs[b]
    return jnp.stack(outs)


# speedup vs baseline: 1.0909x; 1.0110x over previous
import functools

import jax
import jax.numpy as jnp
from jax import lax
from jax.experimental import pallas as pl
from jax.experimental.pallas import tpu as pltpu
from jax.experimental.pallas import tpu_sc as plsc

B, C, H, W = 2, 96, 512, 512
N, K = 8192, 9
HW = H * W
NTAP = 4 * K
CV = C // 16
CP = 128
MOX = 16.0 / W
MOY = 16.0 / H

NC, NS = 2, 16
NW = NC * NS
NPB = N // NW

_HIGH = lax.Precision.HIGHEST


_HB = 32


def _transpose_body(x_ref, o_ref):
    for r in range(_HB):
        o_ref[r, :, 0:C] = x_ref[0, :, r, :].T


def _transpose(feat, b):
    return pl.pallas_call(
        _transpose_body,
        grid=(H // _HB,),
        in_specs=[pl.BlockSpec((1, C, _HB, W), lambda h, _b=b: (_b, 0, h, 0))],
        out_specs=pl.BlockSpec((_HB, W, CP), lambda h: (h, 0, 0)),
        out_shape=jax.ShapeDtypeStruct((H, W, CP), jnp.float32),
    )(feat)


_ACH = 16
_ANCH = NPB // _ACH


def _sc1_body(feat_hbm, xs_hbm, ys_hbm, fa_hbm, xv, yv, idxb, rows,
              outv, sem):
    wid = lax.axis_index("s") * NC + lax.axis_index("c")
    base = wid * NPB
    pltpu.sync_copy(xs_hbm.at[pl.ds(base, NPB)], xv)
    pltpu.sync_copy(ys_hbm.at[pl.ds(base, NPB)], yv)

    def chunk(i, carry):
        px = xv[pl.ds(i * _ACH, 16)]
        py = yv[pl.ds(i * _ACH, 16)]
        fx = (px + 1.0) * 0.5 * (W - 1)
        fy = (py + 1.0) * 0.5 * (H - 1)
        x0 = fx.astype(jnp.int32)
        y0 = fy.astype(jnp.int32)
        wx1 = fx - x0.astype(jnp.float32)
        wy1 = fy - y0.astype(jnp.float32)
        wx0 = 1.0 - wx1
        wy0 = 1.0 - wy1
        x1 = jnp.minimum(x0 + 1, W - 1)
        y1 = jnp.minimum(y0 + 1, H - 1)
        r0 = y0 * W
        r1 = y1 * W
        idxb[pl.ds(0, 16)] = r0 + x0
        idxb[pl.ds(16, 16)] = r0 + x1
        idxb[pl.ds(32, 16)] = r1 + x0
        idxb[pl.ds(48, 16)] = r1 + x1
        w00 = wy0 * wx0
        w01 = wy0 * wx1
        w10 = wy1 * wx0
        w11 = wy1 * wx1
        pltpu.async_copy(feat_hbm.at[idxb], rows, sem).wait()

        for p in range(_ACH):
            w0 = w00[p]
            w1 = w01[p]
            w2 = w10[p]
            w3 = w11[p]
            for cv in range(CV):
                sl = pl.ds(cv * 16, 16)
                outv[i * _ACH + p, sl] = (
                    w0 * rows[p, sl] + w1 * rows[16 + p, sl]
                    + w2 * rows[32 + p, sl] + w3 * rows[48 + p, sl])
        return carry

    lax.fori_loop(0, _ANCH, chunk, 0)
    pltpu.sync_copy(outv, fa_hbm.at[pl.ds(base, NPB)])


@functools.cache
def _sc_anchor_kernel():
    return pl.kernel(
        _sc1_body,
        out_type=jax.ShapeDtypeStruct((N, C), jnp.float32),
        mesh=plsc.VectorSubcoreMesh(core_axis_name="c", subcore_axis_name="s",
                                    num_cores=NC, num_subcores=NS),
        scratch_types=[
            pltpu.VMEM((NPB,), jnp.float32),
            pltpu.VMEM((NPB,), jnp.float32),
            pltpu.VMEM((4 * _ACH,), jnp.int32),
            pltpu.VMEM((4 * _ACH, CP), jnp.float32),
            pltpu.VMEM((NPB, C), jnp.float32),
            pltpu.SemaphoreType.DMA,
        ],
    )


_RB = 2048


def _mlp_body(x_ref, xs_ref, ys_ref, w1_ref, b1_ref, wr_ref, br_ref, w2_ref,
              b2_ref, idx_ref, wts_ref):
    x = x_ref[...]
    h = jnp.dot(x, w1_ref[...], precision=_HIGH) + b1_ref[...]
    h = jnp.where(h >= 0, h, 0.2 * h)
    h2 = h + jnp.dot(h, wr_ref[...], precision=_HIGH) + br_ref[...]
    h2 = jnp.where(h2 >= 0, h2, 0.2 * h2)
    r = jnp.dot(h2, w2_ref[...], precision=_HIGH) + b2_ref[...]

    rt = r.T
    xo = jnp.tanh(rt[0:9]) * MOX
    yo = jnp.tanh(rt[9:18]) * MOY
    wl = rt[18:27]
    m = jnp.max(wl, axis=0, keepdims=True)
    e = jnp.exp(wl - m)
    dynw = e / jnp.sum(e, axis=0, keepdims=True)

    cx = xs_ref[...]
    cy = ys_ref[...]
    fx = (cx + xo + 1.0) * 0.5 * (W - 1)
    fy = (cy + yo + 1.0) * 0.5 * (H - 1)
    x0f = jnp.floor(fx)
    y0f = jnp.floor(fy)
    wx1 = fx - x0f
    wy1 = fy - y0f
    wx0 = 1.0 - wx1
    wy0 = 1.0 - wy1
    x0 = jnp.clip(x0f.astype(jnp.int32), 0, W - 1)
    x1 = jnp.clip(x0f.astype(jnp.int32) + 1, 0, W - 1)
    y0 = jnp.clip(y0f.astype(jnp.int32), 0, H - 1)
    y1 = jnp.clip(y0f.astype(jnp.int32) + 1, 0, H - 1)

    r0 = y0 * W
    r1 = y1 * W
    idx_ref[...] = jnp.concatenate(
        [r0 + x0, r0 + x1, r1 + x0, r1 + x1], axis=0).T
    wts_ref[...] = jnp.concatenate(
        [dynw * wy0 * wx0, dynw * wy0 * wx1,
         dynw * wy1 * wx0, dynw * wy1 * wx1], axis=0).T


def _mlp(xin, xs2, ys2, w1t, b1, wrt, br, w2t, b2p):
    return pl.pallas_call(
        _mlp_body,
        grid=(N // _RB,),
        in_specs=[
            pl.BlockSpec((_RB, 128), lambda g: (g, 0)),
            pl.BlockSpec((1, _RB), lambda g: (0, g)),
            pl.BlockSpec((1, _RB), lambda g: (0, g)),
            pl.BlockSpec((128, 64), lambda g: (0, 0)),
            pl.BlockSpec((1, 64), lambda g: (0, 0)),
            pl.BlockSpec((64, 64), lambda g: (0, 0)),
            pl.BlockSpec((1, 64), lambda g: (0, 0)),
            pl.BlockSpec((64, 32), lambda g: (0, 0)),
            pl.BlockSpec((1, 32), lambda g: (0, 0)),
        ],
        out_specs=[
            pl.BlockSpec((_RB, NTAP), lambda g: (g, 0)),
            pl.BlockSpec((_RB, NTAP), lambda g: (g, 0)),
        ],
        out_shape=[
            jax.ShapeDtypeStruct((N, NTAP), jnp.int32),
            jax.ShapeDtypeStruct((N, NTAP), jnp.float32),
        ],
    )(xin, xs2, ys2, w1t, b1, wrt, br, w2t, b2p)


_PC = 2
_RING = 4
_SEC = 2
_PTS_S = NPB // _SEC
_NCH_S = _PTS_S // _PC


def _sc2_body(feat_hbm, idx_hbm, wts_hbm, out_hbm, idxv, wtsv, rows, outv,
              *sems):
    wid = lax.axis_index("s") * NC + lax.axis_index("c")
    base = wid * NPB

    def start(ch, slot):
        pltpu.async_copy(
            feat_hbm.at[idxv.at[pl.ds(ch * (_PC * NTAP), _PC * NTAP)]],
            rows.at[slot], sems[slot])

    def wait(ch, slot):
        pltpu.make_async_copy(
            feat_hbm.at[idxv.at[pl.ds(ch * (_PC * NTAP), _PC * NTAP)]],
            rows.at[slot], sems[slot]).wait()

    def section(sct, carry):
        sbase = (base + sct * _PTS_S) * NTAP
        pltpu.sync_copy(idx_hbm.at[pl.ds(sbase, _PTS_S * NTAP)], idxv)
        pltpu.sync_copy(wts_hbm.at[pl.ds(sbase, _PTS_S * NTAP)], wtsv)
        for r in range(_RING):
            start(r, r)

        def group(g, carry2):
            for r in range(_RING):
                ch = g * _RING + r
                wait(ch, r)
                for p in range(_PC):
                    ptl = ch * _PC + p
                    o = ptl * NTAP
                    wa = wtsv[pl.ds(o, 16)]
                    wb = wtsv[pl.ds(o + 16, 16)]
                    wc = wtsv[pl.ds(o + 20, 16)]
                    acc = [None] * CV
                    for j in range(NTAP):
                        if j < 16:
                            wj = wa[j]
                        elif j < 32:
                            wj = wb[j - 16]
                        else:
                            wj = wc[j - 20]
                        for cv in range(CV):
                            t = wj * rows[r, p * NTAP + j, pl.ds(cv * 16, 16)]
                            acc[cv] = t if acc[cv] is None else acc[cv] + t
                    for cv in range(CV):
                        outv[sct * _PTS_S + ptl, pl.ds(cv * 16, 16)] = acc[cv]

                @pl.when(ch + _RING < _NCH_S)
                def _():
                    start(ch + _RING, r)
            return carry2

        lax.fori_loop(0, _NCH_S // _RING, group, 0)
        return carry

    lax.fori_loop(0, _SEC, section, 0)
    pltpu.sync_copy(outv, out_hbm.at[pl.ds(base, NPB)])


@functools.cache
def _sc_deform_kernel():
    return pl.kernel(
        _sc2_body,
        out_type=jax.ShapeDtypeStruct((N, C), jnp.float32),
        mesh=plsc.VectorSubcoreMesh(core_axis_name="c", subcore_axis_name="s",
                                    num_cores=NC, num_subcores=NS),
        scratch_types=[
            pltpu.VMEM((_PTS_S * NTAP,), jnp.int32),
            pltpu.VMEM((_PTS_S * NTAP,), jnp.float32),
            pltpu.VMEM((_RING, _PC * NTAP, CP), jnp.float32),
            pltpu.VMEM((NPB, C), jnp.float32),
        ] + [pltpu.SemaphoreType.DMA] * _RING,
    )


_PERM = tuple(range(0, 2 * K, 2)) + tuple(range(1, 2 * K, 2)) \
    + tuple(range(2 * K, 3 * K))


def kernel(feat_map, coords_2d, W1, b1, Wr, br, W2, b2):
    w1t = jnp.pad(W1, ((0, 0), (0, 30))).T
    perm = jnp.array(_PERM, dtype=jnp.int32)
    w2t = jnp.pad(W2[perm], ((0, 5), (0, 0))).T
    b2p = jnp.pad(b2[perm], (0, 5)).reshape(1, 32)
    b1r = b1.reshape(1, 64)
    brr = br.reshape(1, 64)
    wrt = Wr.T

    featT = [None] * B
    outs = [None] * B
    fas = [None] * B
    for b in range(B):
        featT[b] = _transpose(feat_map, b).reshape(HW, CP)
    sc_prev = None
    for b in range(B):
        xy = coords_2d[b]
        xs = xy[:, 0]
        ys = xy[:, 1]
        ft = featT[b]
        if sc_prev is not None:
            ft, _ = lax.optimization_barrier((ft, sc_prev))
        fas[b] = _sc_anchor_kernel()(ft, xs, ys)
        sc_prev = fas[b]
    for b in range(B):
        xy = coords_2d[b]
        xs = xy[:, 0]
        ys = xy[:, 1]
        xin = jnp.concatenate(
            [fas[b], xy, jnp.zeros((N, 30), jnp.float32)], axis=1)
        idx, wts = _mlp(xin, xs.reshape(1, N), ys.reshape(1, N),
                        w1t, b1r, wrt, brr, w2t, b2p)
        idxf = idx.reshape(-1)
        if sc_prev is not None:
            idxf, _ = lax.optimization_barrier((idxf, sc_prev))
        outs[b] = _sc_deform_kernel()(featT[b], idxf, wts.reshape(-1))
        sc_prev = outs[b]
    return jnp.stack(outs)
